# baseline (speedup 1.0000x reference)
import jax
import jax.numpy as jnp
from jax.experimental import pallas as pl
from jax.experimental.pallas import tpu as pltpu

_B, _N, _DIN, _DOUT, _H = 8, 512, 64, 64, 8
_C = _DOUT // _H
_NEG_SLOPE = 0.2


def _gat_graph_kernel(adjT_ref, x_ref, w_ref, asrc_ref, adst_ref, bias_ref,
                      out_ref):
    x = x_ref[0]
    xw = jnp.dot(x, w_ref[...], preferred_element_type=jnp.float32)

    seg = (jax.lax.broadcasted_iota(jnp.int32, (_H * _C, _H), 0) // _C
           == jax.lax.broadcasted_iota(jnp.int32, (_H * _C, _H), 1)
           ).astype(jnp.float32)
    a_dst = jnp.dot(xw * adst_ref[...], seg,
                    preferred_element_type=jnp.float32)
    a_srcT = jax.lax.dot_general(
        seg, xw * asrc_ref[...],
        dimension_numbers=(((0,), (1,)), ((), ())),
        preferred_element_type=jnp.float32)

    adjT = adjT_ref[0]
    row_t = jax.lax.broadcasted_iota(jnp.int32, (_N, _N), 0)
    col_s = jax.lax.broadcasted_iota(jnp.int32, (_N, _N), 1)
    mask = (adjT != 0) | (row_t == col_s)

    for h in range(_H):
        l = a_dst[:, h:h + 1] + a_srcT[h:h + 1, :]
        l = jnp.where(l >= 0, l, _NEG_SLOPE * l)
        l = jnp.where(mask, l, -1e30)
        m = jnp.max(l, axis=1, keepdims=True)
        e = jnp.where(mask, jnp.exp(l - m), 0.0)
        denom = jnp.sum(e, axis=1, keepdims=True)
        attn = e / denom
        out_h = jnp.dot(attn, xw[:, h * _C:(h + 1) * _C],
                        preferred_element_type=jnp.float32)
        y = out_h + bias_ref[0, h * _C:(h + 1) * _C]
        out_ref[0, :, h * _C:(h + 1) * _C] = jnp.where(
            y > 0, y, jnp.exp(jnp.minimum(y, 0.0)) - 1.0)


def kernel(features_batch, adj_mats_batch, W, att_src, att_dst, bias):
    adjT = adj_mats_batch.transpose(0, 2, 1)
    asrc = att_src.reshape(1, _H * _C)
    adst = att_dst.reshape(1, _H * _C)
    bias2 = bias.reshape(1, _DOUT)

    out = pl.pallas_call(
        _gat_graph_kernel,
        grid=(_B,),
        in_specs=[
            pl.BlockSpec((1, _N, _N), lambda b: (b, 0, 0)),
            pl.BlockSpec((1, _N, _DIN), lambda b: (b, 0, 0)),
            pl.BlockSpec((_DIN, _H * _C), lambda b: (0, 0)),
            pl.BlockSpec((1, _H * _C), lambda b: (0, 0)),
            pl.BlockSpec((1, _H * _C), lambda b: (0, 0)),
            pl.BlockSpec((1, _DOUT), lambda b: (0, 0)),
        ],
        out_specs=pl.BlockSpec((1, _N, _DOUT), lambda b: (b, 0, 0)),
        out_shape=jax.ShapeDtypeStruct((_B, _N, _DOUT), jnp.float32),
    )(adjT, features_batch, W, asrc, adst, bias2)
    return out


# speedup vs baseline: 1.7937x; 1.7937x over previous
import jax
import jax.numpy as jnp
from jax.experimental import pallas as pl

_B, _N, _DIN, _DOUT, _H = 8, 512, 64, 64, 8
_C = _DOUT // _H
_NEG_SLOPE = 0.2


def _gat_graph_kernel(adj_ref, x_ref, w_ref, asrc_ref, adst_ref, bias_ref,
                      out_ref):
    x = x_ref[0]
    xw = jnp.dot(x, w_ref[...], preferred_element_type=jnp.float32)

    seg = (jax.lax.broadcasted_iota(jnp.int32, (_H * _C, _H), 0) // _C
           == jax.lax.broadcasted_iota(jnp.int32, (_H * _C, _H), 1)
           ).astype(jnp.float32)
    a_src = jnp.dot(xw * asrc_ref[...], seg,
                    preferred_element_type=jnp.float32)
    a_dstT = jax.lax.dot_general(
        seg, xw * adst_ref[...],
        dimension_numbers=(((0,), (1,)), ((), ())),
        preferred_element_type=jnp.float32)

    u1 = jnp.exp(a_src)
    u2 = jnp.exp(_NEG_SLOPE * a_src)
    v1 = jnp.exp(a_dstT)
    v2 = jnp.exp(_NEG_SLOPE * a_dstT)

    adj = adj_ref[0]
    row_s = jax.lax.broadcasted_iota(jnp.int32, (_N, _N), 0)
    col_t = jax.lax.broadcasted_iota(jnp.int32, (_N, _N), 1)
    mask = (adj != 0) | (row_s == col_t)

    ones_c = jnp.ones((_N, _C), dtype=jnp.float32)
    nums = []
    dens = []
    for h in range(_H):
        e = jnp.maximum(u1[:, h:h + 1] * v1[h:h + 1, :],
                        u2[:, h:h + 1] * v2[h:h + 1, :])
        e = jnp.where(mask, e, 0.0)
        nums.append(jax.lax.dot_general(
            e, xw[:, h * _C:(h + 1) * _C],
            dimension_numbers=(((0,), (0,)), ((), ())),
            preferred_element_type=jnp.float32))
        dens.append(jax.lax.dot_general(
            e, ones_c,
            dimension_numbers=(((0,), (0,)), ((), ())),
            preferred_element_type=jnp.float32))

    num = jnp.concatenate(nums, axis=1)
    den = jnp.concatenate(dens, axis=1)
    y = num / den + bias_ref[...]
    out_ref[0] = jnp.where(y > 0, y, jnp.exp(jnp.minimum(y, 0.0)) - 1.0)


def kernel(features_batch, adj_mats_batch, W, att_src, att_dst, bias):
    asrc = att_src.reshape(1, _H * _C)
    adst = att_dst.reshape(1, _H * _C)
    bias2 = bias.reshape(1, _DOUT)

    out = pl.pallas_call(
        _gat_graph_kernel,
        grid=(_B,),
        in_specs=[
            pl.BlockSpec((1, _N, _N), lambda b: (b, 0, 0)),
            pl.BlockSpec((1, _N, _DIN), lambda b: (b, 0, 0)),
            pl.BlockSpec((_DIN, _H * _C), lambda b: (0, 0)),
            pl.BlockSpec((1, _H * _C), lambda b: (0, 0)),
            pl.BlockSpec((1, _H * _C), lambda b: (0, 0)),
            pl.BlockSpec((1, _DOUT), lambda b: (0, 0)),
        ],
        out_specs=pl.BlockSpec((1, _N, _DOUT), lambda b: (b, 0, 0)),
        out_shape=jax.ShapeDtypeStruct((_B, _N, _DOUT), jnp.float32),
    )(adj_mats_batch, features_batch, W, asrc, adst, bias2)
    return out


# speedup vs baseline: 2.2783x; 1.2702x over previous
import jax
import jax.numpy as jnp
from jax.experimental import pallas as pl

_B, _N, _DIN, _DOUT, _H = 8, 512, 64, 64, 8
_C = _DOUT // _H
_NEG_SLOPE = 0.2


def _gat_graph_kernel(adj_ref, x_ref, w_ref, asrc_ref, adst_ref, bias_ref,
                      out_ref):
    x = x_ref[0]
    xw = jnp.dot(x, w_ref[...], preferred_element_type=jnp.float32)

    seg = (jax.lax.broadcasted_iota(jnp.int32, (_H * _C, _H), 0) // _C
           == jax.lax.broadcasted_iota(jnp.int32, (_H * _C, _H), 1)
           ).astype(jnp.float32)
    a_src = jnp.dot(xw * asrc_ref[...], seg,
                    preferred_element_type=jnp.float32)
    a_dstT = jax.lax.dot_general(
        seg, xw * adst_ref[...],
        dimension_numbers=(((0,), (1,)), ((), ())),
        preferred_element_type=jnp.float32)

    u1 = jnp.exp(a_src).astype(jnp.bfloat16)
    u2 = jnp.exp(_NEG_SLOPE * a_src).astype(jnp.bfloat16)
    v1 = jnp.exp(a_dstT).astype(jnp.bfloat16)
    v2 = jnp.exp(_NEG_SLOPE * a_dstT).astype(jnp.bfloat16)
    xwb = xw.astype(jnp.bfloat16)

    adj = adj_ref[0]
    row_s = jax.lax.broadcasted_iota(jnp.int32, (_N, _N), 0)
    col_t = jax.lax.broadcasted_iota(jnp.int32, (_N, _N), 1)
    mask = (adj != 0) | (row_s == col_t)

    ones_c = jnp.ones((_N, _C), dtype=jnp.bfloat16)
    nums = []
    dens = []
    for h in range(_H):
        e = jnp.maximum(u1[:, h:h + 1] * v1[h:h + 1, :],
                        u2[:, h:h + 1] * v2[h:h + 1, :])
        e = jnp.where(mask, e, jnp.bfloat16(0.0))
        nums.append(jax.lax.dot_general(
            e, xwb[:, h * _C:(h + 1) * _C],
            dimension_numbers=(((0,), (0,)), ((), ())),
            preferred_element_type=jnp.float32))
        dens.append(jax.lax.dot_general(
            e, ones_c,
            dimension_numbers=(((0,), (0,)), ((), ())),
            preferred_element_type=jnp.float32))

    num = jnp.concatenate(nums, axis=1)
    den = jnp.concatenate(dens, axis=1)
    y = num / den + bias_ref[...]
    out_ref[0] = jnp.where(y > 0, y, jnp.exp(jnp.minimum(y, 0.0)) - 1.0)


def kernel(features_batch, adj_mats_batch, W, att_src, att_dst, bias):
    asrc = att_src.reshape(1, _H * _C)
    adst = att_dst.reshape(1, _H * _C)
    bias2 = bias.reshape(1, _DOUT)

    out = pl.pallas_call(
        _gat_graph_kernel,
        grid=(_B,),
        in_specs=[
            pl.BlockSpec((1, _N, _N), lambda b: (b, 0, 0)),
            pl.BlockSpec((1, _N, _DIN), lambda b: (b, 0, 0)),
            pl.BlockSpec((_DIN, _H * _C), lambda b: (0, 0)),
            pl.BlockSpec((1, _H * _C), lambda b: (0, 0)),
            pl.BlockSpec((1, _H * _C), lambda b: (0, 0)),
            pl.BlockSpec((1, _DOUT), lambda b: (0, 0)),
        ],
        out_specs=pl.BlockSpec((1, _N, _DOUT), lambda b: (b, 0, 0)),
        out_shape=jax.ShapeDtypeStruct((_B, _N, _DOUT), jnp.float32),
    )(adj_mats_batch, features_batch, W, asrc, adst, bias2)
    return out


# speedup vs baseline: 2.3413x; 1.0276x over previous
import jax
import jax.numpy as jnp
from jax.experimental import pallas as pl

_B, _N, _DIN, _DOUT, _H = 8, 512, 64, 64, 8
_C = _DOUT // _H
_NEG_SLOPE = 0.2


def _gat_graph_kernel(adj_ref, x_ref, w_ref, par_ref, out_ref):
    x = x_ref[...]
    xw = jnp.dot(x, w_ref[...], preferred_element_type=jnp.float32)

    seg = (jax.lax.broadcasted_iota(jnp.int32, (_H * _C, _H), 0) // _C
           == jax.lax.broadcasted_iota(jnp.int32, (_H * _C, _H), 1)
           ).astype(jnp.float32)
    a_src = jnp.dot(xw * par_ref[0:1, :], seg,
                    preferred_element_type=jnp.float32)
    a_dstT = jax.lax.dot_general(
        seg, xw * par_ref[1:2, :],
        dimension_numbers=(((0,), (1,)), ((), ())),
        preferred_element_type=jnp.float32)

    u1 = a_src.astype(jnp.bfloat16)
    v1 = a_dstT.astype(jnp.bfloat16)
    xwb = xw.astype(jnp.bfloat16)

    adj = adj_ref[...]
    row_s = jax.lax.broadcasted_iota(jnp.int32, (_N, _N), 0)
    col_t = jax.lax.broadcasted_iota(jnp.int32, (_N, _N), 1)
    mask = (adj != 0) | (row_s == col_t)

    ones_c = jnp.ones((_N, _C), dtype=jnp.bfloat16)
    nums = []
    dens = []
    for h in range(_H):
        l = u1[:, h:h + 1] + v1[h:h + 1, :]
        e = jnp.exp(jnp.maximum(l, jnp.bfloat16(_NEG_SLOPE) * l))
        e = jnp.where(mask, e, jnp.bfloat16(0.0))
        nums.append(jax.lax.dot_general(
            e, xwb[:, h * _C:(h + 1) * _C],
            dimension_numbers=(((0,), (0,)), ((), ())),
            preferred_element_type=jnp.float32))
        dens.append(jax.lax.dot_general(
            e, ones_c,
            dimension_numbers=(((0,), (0,)), ((), ())),
            preferred_element_type=jnp.float32))

    num = jnp.concatenate(nums, axis=1)
    den = jnp.concatenate(dens, axis=1)
    y = num / den + par_ref[2:3, :]
    out_ref[...] = jnp.where(y > 0, y, jnp.exp(jnp.minimum(y, 0.0)) - 1.0)


def kernel(features_batch, adj_mats_batch, W, att_src, att_dst, bias):
    params = jnp.stack([att_src.reshape(_H * _C),
                        att_dst.reshape(_H * _C), bias])
    adj2 = adj_mats_batch.reshape(_B * _N, _N)
    x2 = features_batch.reshape(_B * _N, _DIN)

    out = pl.pallas_call(
        _gat_graph_kernel,
        grid=(_B,),
        in_specs=[
            pl.BlockSpec((_N, _N), lambda b: (b, 0)),
            pl.BlockSpec((_N, _DIN), lambda b: (b, 0)),
            pl.BlockSpec((_DIN, _H * _C), lambda b: (0, 0)),
            pl.BlockSpec((3, _H * _C), lambda b: (0, 0)),
        ],
        out_specs=pl.BlockSpec((_N, _DOUT), lambda b: (b, 0)),
        out_shape=jax.ShapeDtypeStruct((_B * _N, _DOUT), jnp.float32),
    )(adj2, x2, W, params)
    return out.reshape(_B, _N, _DOUT)


# speedup vs baseline: 2.7810x; 1.1878x over previous
import jax
import jax.numpy as jnp
from jax.experimental import pallas as pl

_B, _N, _DIN, _DOUT, _H = 8, 512, 64, 64, 8
_C = _DOUT // _H
_NEG_SLOPE = 0.2


def _gat_graph_kernel(adj_ref, x_ref, w_ref, par_ref, out_ref):
    xt = x_ref[0]
    xw = jax.lax.dot_general(
        xt, w_ref[...],
        dimension_numbers=(((0,), (0,)), ((), ())),
        preferred_element_type=jnp.float32)

    seg = (jax.lax.broadcasted_iota(jnp.int32, (_H * _C, _H), 0) // _C
           == jax.lax.broadcasted_iota(jnp.int32, (_H * _C, _H), 1)
           ).astype(jnp.float32)
    a_src = jnp.dot(xw * par_ref[0:1, :], seg,
                    preferred_element_type=jnp.float32)
    a_dstT = jax.lax.dot_general(
        seg, xw * par_ref[1:2, :],
        dimension_numbers=(((0,), (1,)), ((), ())),
        preferred_element_type=jnp.float32)

    u1 = a_src.astype(jnp.bfloat16)
    v1 = a_dstT.astype(jnp.bfloat16)
    xwb = xw.astype(jnp.bfloat16)

    adj = adj_ref[0]
    row_s = jax.lax.broadcasted_iota(jnp.int32, (_N, _N), 0)
    col_t = jax.lax.broadcasted_iota(jnp.int32, (_N, _N), 1)
    mask = (adj != 0) | (row_s == col_t)

    ones_c = jnp.ones((_N, _C), dtype=jnp.bfloat16)
    mms = []
    for h in range(_H):
        l = u1[:, h:h + 1] + v1[h:h + 1, :]
        e = jnp.exp(jnp.maximum(l, jnp.bfloat16(_NEG_SLOPE) * l))
        e = jnp.where(mask, e, jnp.bfloat16(0.0))
        g = jnp.concatenate([xwb[:, h * _C:(h + 1) * _C], ones_c], axis=1)
        mms.append(jax.lax.dot_general(
            e, g,
            dimension_numbers=(((0,), (0,)), ((), ())),
            preferred_element_type=jnp.float32))

    num = jnp.concatenate([m[:, :_C] for m in mms], axis=1)
    den = jnp.concatenate([m[:, _C:] for m in mms], axis=1)
    y = num / den + par_ref[2:3, :]
    y = jnp.where(y > 0, y, jnp.exp(jnp.minimum(y, 0.0)) - 1.0)
    out_ref[0] = y.T


def kernel(features_batch, adj_mats_batch, W, att_src, att_dst, bias):
    params = jnp.stack([att_src.reshape(_H * _C),
                        att_dst.reshape(_H * _C), bias])
    xt = features_batch.transpose(0, 2, 1)

    out = pl.pallas_call(
        _gat_graph_kernel,
        grid=(_B,),
        in_specs=[
            pl.BlockSpec((1, _N, _N), lambda b: (b, 0, 0)),
            pl.BlockSpec((1, _DIN, _N), lambda b: (b, 0, 0)),
            pl.BlockSpec((_DIN, _H * _C), lambda b: (0, 0)),
            pl.BlockSpec((3, _H * _C), lambda b: (0, 0)),
        ],
        out_specs=pl.BlockSpec((1, _DOUT, _N), lambda b: (b, 0, 0)),
        out_shape=jax.ShapeDtypeStruct((_B, _DOUT, _N), jnp.float32),
    )(adj_mats_batch, xt, W, params)
    return out.transpose(0, 2, 1)


# speedup vs baseline: 2.9993x; 1.0785x over previous
import jax
import jax.numpy as jnp
from jax.experimental import pallas as pl

_B, _N, _DIN, _DOUT, _H = 8, 512, 64, 64, 8
_C = _DOUT // _H
_NEG_SLOPE = 0.2


_LOG2E = 1.4426950408889634


def _gat_graph_kernel(adj_ref, x_ref, w_ref, asrc_ref, adst_ref, bias_ref,
                      out_ref):
    xt = x_ref[0]
    xw = jax.lax.dot_general(
        xt, w_ref[...],
        dimension_numbers=(((0,), (0,)), ((), ())),
        preferred_element_type=jnp.float32)
    asrc = asrc_ref[...] * _LOG2E
    adst = adst_ref[...] * _LOG2E

    seg = (jax.lax.broadcasted_iota(jnp.int32, (_H * _C, _H), 0) // _C
           == jax.lax.broadcasted_iota(jnp.int32, (_H * _C, _H), 1)
           ).astype(jnp.float32)
    colsel = (jax.lax.broadcasted_iota(jnp.int32, (_H * _C, _C), 0) % _C
              == jax.lax.broadcasted_iota(jnp.int32, (_H * _C, _C), 1)
              ).astype(jnp.float32)
    ones_c1 = jnp.ones((_C, 1), dtype=jnp.float32)

    def _blockdiag(att):
        tmp = jnp.dot(seg, att, preferred_element_type=jnp.float32)
        flat = jnp.dot(tmp * colsel, ones_c1,
                       preferred_element_type=jnp.float32)
        return seg * flat

    a_src = jnp.dot(xw, _blockdiag(asrc),
                    preferred_element_type=jnp.float32)
    a_dstT = jax.lax.dot_general(
        _blockdiag(adst), xw,
        dimension_numbers=(((0,), (1,)), ((), ())),
        preferred_element_type=jnp.float32)

    u1 = a_src.astype(jnp.bfloat16)
    v1 = a_dstT.astype(jnp.bfloat16)
    xwb = xw.astype(jnp.bfloat16)

    adj = adj_ref[0]
    row_s = jax.lax.broadcasted_iota(jnp.int32, (_N, _N), 0)
    col_t = jax.lax.broadcasted_iota(jnp.int32, (_N, _N), 1)
    mask = (adj != 0) | (row_s == col_t)

    ones_c = jnp.ones((_N, _C), dtype=jnp.bfloat16)
    mms = []
    for h in range(_H):
        l = u1[:, h:h + 1] + v1[h:h + 1, :]
        e = jnp.exp2(jnp.maximum(l, jnp.bfloat16(_NEG_SLOPE) * l))
        e = jnp.where(mask, e, jnp.bfloat16(0.0))
        g = jnp.concatenate([xwb[:, h * _C:(h + 1) * _C], ones_c], axis=1)
        mms.append(jax.lax.dot_general(
            e, g,
            dimension_numbers=(((0,), (0,)), ((), ())),
            preferred_element_type=jnp.float32))

    num = jnp.concatenate([m[:, :_C] for m in mms], axis=1)
    den = jnp.concatenate([m[:, _C:] for m in mms], axis=1)
    y = num / den + bias_ref[...][None, :]
    y = jnp.where(y > 0, y,
                  jnp.exp2(jnp.minimum(y, 0.0) * _LOG2E) - 1.0)
    out_ref[0] = y.T


def kernel(features_batch, adj_mats_batch, W, att_src, att_dst, bias):
    xt = features_batch.transpose(0, 2, 1)

    out = pl.pallas_call(
        _gat_graph_kernel,
        grid=(_B,),
        in_specs=[
            pl.BlockSpec((1, _N, _N), lambda b: (b, 0, 0)),
            pl.BlockSpec((1, _DIN, _N), lambda b: (b, 0, 0)),
            pl.BlockSpec((_DIN, _H * _C), lambda b: (0, 0)),
            pl.BlockSpec((_H, _C), lambda b: (0, 0)),
            pl.BlockSpec((_H, _C), lambda b: (0, 0)),
            pl.BlockSpec((_DOUT,), lambda b: (0,)),
        ],
        out_specs=pl.BlockSpec((1, _DOUT, _N), lambda b: (b, 0, 0)),
        out_shape=jax.ShapeDtypeStruct((_B, _DOUT, _N), jnp.float32),
    )(adj_mats_batch, xt, W, att_src, att_dst, bias)
    return out.transpose(0, 2, 1)


# speedup vs baseline: 4.6332x; 1.5448x over previous
import jax
import jax.numpy as jnp
from jax.experimental import pallas as pl

_B, _N, _DIN, _DOUT, _H = 8, 512, 64, 64, 8
_C = _DOUT // _H
_NEG_SLOPE = 0.2


_LOG2E = 1.4426950408889634


def _gat_graph_kernel(adj_ref, x_ref, w_ref, asrc_ref, adst_ref, bias_ref,
                      out_ref):
    xt = x_ref[0]
    xw = jax.lax.dot_general(
        xt, w_ref[...],
        dimension_numbers=(((0,), (0,)), ((), ())),
        preferred_element_type=jnp.float32)
    asrc = asrc_ref[...] * _LOG2E
    adst = adst_ref[...] * _LOG2E

    seg = (jax.lax.broadcasted_iota(jnp.int32, (_H * _C, _H), 0) // _C
           == jax.lax.broadcasted_iota(jnp.int32, (_H * _C, _H), 1)
           ).astype(jnp.float32)
    colsel = (jax.lax.broadcasted_iota(jnp.int32, (_H * _C, _C), 0) % _C
              == jax.lax.broadcasted_iota(jnp.int32, (_H * _C, _C), 1)
              ).astype(jnp.float32)
    ones_c1 = jnp.ones((_C, 1), dtype=jnp.float32)

    def _blockdiag(att):
        tmp = jnp.dot(seg, att, preferred_element_type=jnp.float32)
        flat = jnp.dot(tmp * colsel, ones_c1,
                       preferred_element_type=jnp.float32)
        return seg * flat

    a_src = jnp.dot(xw, _blockdiag(asrc),
                    preferred_element_type=jnp.float32)
    a_dstT = jax.lax.dot_general(
        _blockdiag(adst), xw,
        dimension_numbers=(((0,), (1,)), ((), ())),
        preferred_element_type=jnp.float32)

    u1 = a_src.astype(jnp.bfloat16)
    v1 = a_dstT.astype(jnp.bfloat16)
    xwb = xw.astype(jnp.bfloat16)

    adj = adj_ref[0]
    row_s = jax.lax.broadcasted_iota(jnp.int32, (_N, _N), 0)
    col_t = jax.lax.broadcasted_iota(jnp.int32, (_N, _N), 1)
    mask = (adj != 0) | (row_s == col_t)

    ones_c = jnp.ones((_N, _C), dtype=jnp.bfloat16)
    mms = []
    for h in range(_H):
        l = u1[:, h:h + 1] + v1[h:h + 1, :]
        e = jnp.exp2(jnp.maximum(l, jnp.bfloat16(_NEG_SLOPE) * l))
        e = jnp.where(mask, e, jnp.bfloat16(0.0))
        g = jnp.concatenate([xwb[:, h * _C:(h + 1) * _C], ones_c], axis=1)
        mms.append(jax.lax.dot_general(
            g, e,
            dimension_numbers=(((0,), (0,)), ((), ())),
            preferred_element_type=jnp.float32))

    num = jnp.concatenate([m[:_C] for m in mms], axis=0)
    den = jnp.concatenate([m[_C:] for m in mms], axis=0)
    y = num / den + bias_ref[...][:, None]
    y = jnp.where(y > 0, y,
                  jnp.exp2(jnp.minimum(y, 0.0) * _LOG2E) - 1.0)
    out_ref[0] = y


def kernel(features_batch, adj_mats_batch, W, att_src, att_dst, bias):
    xt = features_batch.transpose(0, 2, 1)

    out = pl.pallas_call(
        _gat_graph_kernel,
        grid=(_B,),
        in_specs=[
            pl.BlockSpec((1, _N, _N), lambda b: (b, 0, 0)),
            pl.BlockSpec((1, _DIN, _N), lambda b: (b, 0, 0)),
            pl.BlockSpec((_DIN, _H * _C), lambda b: (0, 0)),
            pl.BlockSpec((_H, _C), lambda b: (0, 0)),
            pl.BlockSpec((_H, _C), lambda b: (0, 0)),
            pl.BlockSpec((_DOUT,), lambda b: (0,)),
        ],
        out_specs=pl.BlockSpec((1, _DOUT, _N), lambda b: (b, 0, 0)),
        out_shape=jax.ShapeDtypeStruct((_B, _DOUT, _N), jnp.float32),
    )(adj_mats_batch, xt, W, att_src, att_dst, bias)
    return out.transpose(0, 2, 1)


# speedup vs baseline: 5.0891x; 1.0984x over previous
import jax
import jax.numpy as jnp
from jax.experimental import pallas as pl

_B, _N, _DIN, _DOUT, _H = 8, 512, 64, 64, 8
_C = _DOUT // _H
_NEG_SLOPE = 0.2


_LOG2E = 1.4426950408889634


_GPP = 2


def _gat_graph_kernel(adj_ref, x_ref, w_ref, asrc_ref, adst_ref, bias_ref,
                      out_ref):
    asrc = asrc_ref[...] * _LOG2E
    adst = adst_ref[...] * _LOG2E

    seg = (jax.lax.broadcasted_iota(jnp.int32, (_H * _C, _H), 0) // _C
           == jax.lax.broadcasted_iota(jnp.int32, (_H * _C, _H), 1)
           ).astype(jnp.float32)
    colsel = (jax.lax.broadcasted_iota(jnp.int32, (_H * _C, _C), 0) % _C
              == jax.lax.broadcasted_iota(jnp.int32, (_H * _C, _C), 1)
              ).astype(jnp.float32)
    ones_c1 = jnp.ones((_C, 1), dtype=jnp.float32)

    def _blockdiag(att):
        tmp = jnp.dot(seg, att, preferred_element_type=jnp.float32)
        flat = jnp.dot(tmp * colsel, ones_c1,
                       preferred_element_type=jnp.float32)
        return seg * flat

    bd_src = _blockdiag(asrc)
    bd_dst = _blockdiag(adst)
    row_s = jax.lax.broadcasted_iota(jnp.int32, (_N, _N), 0)
    col_t = jax.lax.broadcasted_iota(jnp.int32, (_N, _N), 1)
    eye = row_s == col_t
    ones_c = jnp.ones((_N, _C), dtype=jnp.bfloat16)

    for i in range(_GPP):
        xt = x_ref[i]
        xw = jax.lax.dot_general(
            xt, w_ref[...],
            dimension_numbers=(((0,), (0,)), ((), ())),
            preferred_element_type=jnp.float32)
        a_src = jnp.dot(xw, bd_src,
                        preferred_element_type=jnp.float32)
        a_dstT = jax.lax.dot_general(
            bd_dst, xw,
            dimension_numbers=(((0,), (1,)), ((), ())),
            preferred_element_type=jnp.float32)

        u1 = a_src.astype(jnp.bfloat16)
        v1 = a_dstT.astype(jnp.bfloat16)
        xwb = xw.astype(jnp.bfloat16)

        mask = (adj_ref[i] != 0) | eye

        mms = []
        for h in range(_H):
            l = u1[:, h:h + 1] + v1[h:h + 1, :]
            e = jnp.exp2(jnp.maximum(l, jnp.bfloat16(_NEG_SLOPE) * l))
            e = jnp.where(mask, e, jnp.bfloat16(0.0))
            g = jnp.concatenate([xwb[:, h * _C:(h + 1) * _C], ones_c],
                                axis=1)
            mms.append(jax.lax.dot_general(
                g, e,
                dimension_numbers=(((0,), (0,)), ((), ())),
                preferred_element_type=jnp.float32))

        num = jnp.concatenate([m[:_C] for m in mms], axis=0)
        den = jnp.concatenate([m[_C:] for m in mms], axis=0)
        y = num / den + bias_ref[...][:, None]
        y = jnp.where(y > 0, y,
                      jnp.exp2(jnp.minimum(y, 0.0) * _LOG2E) - 1.0)
        out_ref[i] = y


def kernel(features_batch, adj_mats_batch, W, att_src, att_dst, bias):
    xt = features_batch.transpose(0, 2, 1)

    out = pl.pallas_call(
        _gat_graph_kernel,
        grid=(_B // _GPP,),
        in_specs=[
            pl.BlockSpec((_GPP, _N, _N), lambda b: (b, 0, 0)),
            pl.BlockSpec((_GPP, _DIN, _N), lambda b: (b, 0, 0)),
            pl.BlockSpec((_DIN, _H * _C), lambda b: (0, 0)),
            pl.BlockSpec((_H, _C), lambda b: (0, 0)),
            pl.BlockSpec((_H, _C), lambda b: (0, 0)),
            pl.BlockSpec((_DOUT,), lambda b: (0,)),
        ],
        out_specs=pl.BlockSpec((_GPP, _DOUT, _N), lambda b: (b, 0, 0)),
        out_shape=jax.ShapeDtypeStruct((_B, _DOUT, _N), jnp.float32),
    )(adj_mats_batch, xt, W, att_src, att_dst, bias)
    return out.transpose(0, 2, 1)
